# baseline (speedup 1.0000x reference)
import functools

import jax
import jax.numpy as jnp
from jax import lax
from jax.experimental import pallas as pl
from jax.experimental.pallas import tpu as pltpu
from jax.experimental.pallas import tpu_sc as plsc

_C = 5
_S = 4096
_Q = 8
_LANES = 16


def _compute_block(seq_v, out_v, q, pb):
    def j_body(j, carry):
        s = seq_v[pl.ds(q * _S + j * _LANES, _LANES)]
        for c in range(_C):
            out_v[pb, c, pl.ds(j * _LANES, _LANES)] = jnp.where(
                s == c, 1.0, 0.0
            ).astype(jnp.float32)
        return carry

    lax.fori_loop(0, _S // _LANES, j_body, 0, unroll=8)


def _sc_body(L, NW, stride, seq_hbm, out_hbm, seq_v, out_v, sem0, sem1):
    wid = lax.axis_index("s") * 2 + lax.axis_index("c")
    span = _Q * _S
    aligned = (L // 128) * 128
    base = pl.multiple_of(jnp.minimum(wid * stride, aligned - span), 128)
    sem = (sem0, sem1)

    pltpu.sync_copy(seq_hbm.at[pl.ds(base, span)], seq_v)

    out_cps = [[], []]
    for q in range(_Q):
        pb = q % 2
        for cp in out_cps[pb]:
            cp.wait()
        _compute_block(seq_v, out_v, q, pb)
        col0 = pl.multiple_of(base + q * _S, 128)
        out_cps[pb] = [
            pltpu.async_copy(
                out_v.at[pb],
                out_hbm.at[:, pl.ds(col0, _S)],
                sem[pb],
            )
        ]
    for pb in range(2):
        for cp in out_cps[pb]:
            cp.wait()


def _tail_body(seq_ref, _sc_ref, out_ref):
    s = seq_ref[:]
    classes = jax.lax.broadcasted_iota(jnp.int32, (_C, 128), 0)
    out_ref[:, :] = (s[None, :] == classes).astype(jnp.float32)


def kernel(seq):
    L = seq.shape[0]
    NW = 32
    span = _Q * _S
    aligned = (L // 128) * 128
    stride = -(-(aligned - span) // (NW - 1))
    stride = -(-stride // 128) * 128
    assert stride <= span
    mesh = plsc.VectorSubcoreMesh(core_axis_name="c", subcore_axis_name="s")
    body = functools.partial(_sc_body, L, NW, stride)
    sc_out = pl.kernel(
        body,
        mesh=mesh,
        compiler_params=pltpu.CompilerParams(use_tc_tiling_on_sc=True),
        out_type=jax.ShapeDtypeStruct((_C, L), jnp.float32),
        scratch_types=[
            pltpu.VMEM((span,), jnp.int32),
            pltpu.VMEM((2, _C, _S), jnp.float32),
            pltpu.SemaphoreType.DMA,
            pltpu.SemaphoreType.DMA,
        ],
    )(seq)
    if L % 128 == 0:
        return sc_out
    j = L // 128
    return pl.pallas_call(
        _tail_body,
        grid=(1,),
        in_specs=[
            pl.BlockSpec((128,), lambda i: (j,)),
            pl.BlockSpec(memory_space=pl.ANY),
        ],
        out_specs=pl.BlockSpec((_C, 128), lambda i: (0, j)),
        out_shape=jax.ShapeDtypeStruct((_C, L), jnp.float32),
        input_output_aliases={1: 0},
    )(seq, sc_out)


# speedup vs baseline: 1.1135x; 1.1135x over previous
import functools

import jax
import jax.numpy as jnp
from jax import lax
from jax.experimental import pallas as pl
from jax.experimental.pallas import tpu as pltpu
from jax.experimental.pallas import tpu_sc as plsc

_C = 5
_S = 4096
_Q = 8
_LANES = 16


def _compute_block(seq_v, out_v, q, pb):
    def j_body(j, carry):
        s = seq_v[pl.ds(q * _S + j * _LANES, _LANES)]
        for c in range(_C):
            out_v[pb, c, pl.ds(j * _LANES, _LANES)] = jnp.where(
                s == c, 1.0, 0.0
            ).astype(jnp.float32)
        return carry

    lax.fori_loop(0, _S // _LANES, j_body, 0, unroll=4)


def _sc_body(L, NW, stride, seq_hbm, out_hbm, seq_v, out_v, sem0, sem1):
    wid = lax.axis_index("s") * 2 + lax.axis_index("c")
    span = _Q * _S
    aligned = (L // 128) * 128
    base = pl.multiple_of(jnp.minimum(wid * stride, aligned - span), 128)
    sem = (sem0, sem1)

    pltpu.sync_copy(seq_hbm.at[pl.ds(base, span)], seq_v)

    out_cps = [[], []]
    for q in range(_Q):
        pb = q % 2
        for cp in out_cps[pb]:
            cp.wait()
        _compute_block(seq_v, out_v, q, pb)
        col0 = pl.multiple_of(base + q * _S, 128)
        out_cps[pb] = [
            pltpu.async_copy(
                out_v.at[pb],
                out_hbm.at[:, pl.ds(col0, _S)],
                sem[pb],
            )
        ]
    for pb in range(2):
        for cp in out_cps[pb]:
            cp.wait()


def _tail_body(seq_ref, _sc_ref, out_ref):
    s = seq_ref[:]
    classes = jax.lax.broadcasted_iota(jnp.int32, (_C, 128), 0)
    out_ref[:, :] = (s[None, :] == classes).astype(jnp.float32)


def kernel(seq):
    L = seq.shape[0]
    NW = 32
    span = _Q * _S
    aligned = (L // 128) * 128
    stride = -(-(aligned - span) // (NW - 1))
    stride = -(-stride // 128) * 128
    assert stride <= span
    mesh = plsc.VectorSubcoreMesh(core_axis_name="c", subcore_axis_name="s")
    body = functools.partial(_sc_body, L, NW, stride)
    sc_out = pl.kernel(
        body,
        mesh=mesh,
        compiler_params=pltpu.CompilerParams(use_tc_tiling_on_sc=True),
        out_type=jax.ShapeDtypeStruct((_C, L), jnp.float32),
        scratch_types=[
            pltpu.VMEM((span,), jnp.int32),
            pltpu.VMEM((2, _C, _S), jnp.float32),
            pltpu.SemaphoreType.DMA,
            pltpu.SemaphoreType.DMA,
        ],
    )(seq)
    if L % 128 == 0:
        return sc_out
    j = L // 128
    return pl.pallas_call(
        _tail_body,
        grid=(1,),
        in_specs=[
            pl.BlockSpec((128,), lambda i: (j,)),
            pl.BlockSpec(memory_space=pl.ANY),
        ],
        out_specs=pl.BlockSpec((_C, 128), lambda i: (0, j)),
        out_shape=jax.ShapeDtypeStruct((_C, L), jnp.float32),
        input_output_aliases={1: 0},
    )(seq, sc_out)
